# Initial kernel scaffold; baseline (speedup 1.0000x reference)
import jax
import jax.numpy as jnp
from jax.experimental import pallas as pl


def kernel(x, edge_index, batch, W1, as1, ad1, b1, W2, as2, ad2, b2, Wih0, Whh0, bih0, bhh0, Wih1, Whh1, bih1, bhh1, Wh1, bh1, Wh2, bh2):
    raise NotImplementedError("write your pallas kernel here")


# speedup vs baseline: 1.0978x; 1.0978x over previous
import functools

import jax
import jax.numpy as jnp
from jax.experimental import pallas as pl
from jax.experimental.pallas import tpu as pltpu

N_NODES = 50000
N_FRAMES = 1600
N_CHAINS = 50
CHAIN_LEN = 32
HID = 48
HEADS = 4
GRU_DIM = 64
FRAME_DIM = 2 * HID
C_PAD = 56


def _gru_head_body(xseq, wi0, wh0, bi0, bh0, wi1, wh1, bi1, bh1,
                   w1, b1, w2, b2, out):
    G = GRU_DIM

    def gates(gi, gh):
        r = jax.nn.sigmoid(gi[:, :G] + gh[:, :G])
        z = jax.nn.sigmoid(gi[:, G:2 * G] + gh[:, G:2 * G])
        n = jnp.tanh(gi[:, 2 * G:] + r * gh[:, 2 * G:3 * G])
        return (1.0 - z) * n + z * gh[:, 3 * G:]

    def step(t, carry):
        h0, h1 = carry
        x_t = xseq[t]
        gi0 = jnp.dot(x_t, wi0[...], preferred_element_type=jnp.float32) + bi0[...]
        gh0 = jnp.dot(h0, wh0[...], preferred_element_type=jnp.float32) + bh0[...]
        h0n = gates(gi0, jnp.concatenate([gh0, h0], axis=1))
        gi1 = jnp.dot(h0n, wi1[...], preferred_element_type=jnp.float32) + bi1[...]
        gh1 = jnp.dot(h1, wh1[...], preferred_element_type=jnp.float32) + bh1[...]
        h1n = gates(gi1, jnp.concatenate([gh1, h1], axis=1))
        return h0n, h1n

    h0 = jnp.zeros((C_PAD, G), jnp.float32)
    h0, h1 = jax.lax.fori_loop(0, CHAIN_LEN, step, (h0, h0))
    z = jax.nn.relu(jnp.dot(h1, w1[...], preferred_element_type=jnp.float32) + b1[...])
    out[...] = jnp.dot(z, w2[...], preferred_element_type=jnp.float32) + b2[...]


def _gru_head(xseq, Wih0, Whh0, bih0, bhh0, Wih1, Whh1, bih1, bhh1,
              Wh1, bh1, Wh2, bh2):
    args = (xseq, Wih0.T, Whh0.T, bih0[None], bhh0[None], Wih1.T, Whh1.T,
            bih1[None], bhh1[None], Wh1.T, bh1[None], Wh2.T, bh2[None])
    return pl.pallas_call(
        _gru_head_body,
        out_shape=jax.ShapeDtypeStruct((C_PAD, 1), jnp.float32),
    )(*args)


def _gat_conv(x, src, dst, W, att_s, att_d, bias, heads, out_dim, concat):
    N = x.shape[0]
    h = (x @ W).reshape(N, heads, out_dim)
    a_s = (h * att_s[None, :, :]).sum(-1)
    a_d = (h * att_d[None, :, :]).sum(-1)
    e = a_s[src] + a_d[dst]
    e = jnp.where(e > 0, e, 0.2 * e)
    ex = jnp.exp(e)
    denom = jax.ops.segment_sum(ex, dst, num_segments=N)
    alpha = ex / (denom[dst] + 1e-16)
    out = jax.ops.segment_sum(h[src] * alpha[..., None], dst, num_segments=N)
    if concat:
        out = out.reshape(N, heads * out_dim)
    else:
        out = out.mean(axis=1)
    return out + bias


def kernel(x, edge_index, batch, W1, as1, ad1, b1, W2, as2, ad2, b2,
           Wih0, Whh0, bih0, bhh0, Wih1, Whh1, bih1, bhh1, Wh1, bh1, Wh2, bh2):
    N = x.shape[0]
    loop = jnp.arange(N, dtype=edge_index.dtype)
    src = jnp.concatenate([edge_index[0], loop])
    dst = jnp.concatenate([edge_index[1], loop])
    h = jax.nn.elu(_gat_conv(x, src, dst, W1, as1, ad1, b1, HEADS, HID, True))
    h = jax.nn.elu(_gat_conv(h, src, dst, W2, as2, ad2, b2, 1, HID, False))
    s = jax.ops.segment_sum(h, batch, num_segments=N_FRAMES)
    cnt = jax.ops.segment_sum(jnp.ones((N, 1), h.dtype), batch, num_segments=N_FRAMES)
    mean_p = s / jnp.maximum(cnt, 1.0)
    max_p = jax.ops.segment_max(h, batch, num_segments=N_FRAMES)
    max_p = jnp.where(jnp.isfinite(max_p), max_p, 0.0)
    frames = jnp.concatenate([mean_p, max_p], axis=-1)
    seqs = frames.reshape(N_CHAINS, CHAIN_LEN, FRAME_DIM)
    xseq = jnp.transpose(seqs, (1, 0, 2))
    xseq = jnp.pad(xseq, ((0, 0), (0, C_PAD - N_CHAINS), (0, 0)))
    logits = _gru_head(xseq, Wih0, Whh0, bih0, bhh0, Wih1, Whh1,
                       bih1, bhh1, Wh1, bh1, Wh2, bh2)
    return logits[:N_CHAINS, 0]


# speedup vs baseline: 10.2717x; 9.3565x over previous
import functools

import jax
import jax.numpy as jnp
from jax import lax
from jax.experimental import pallas as pl
from jax.experimental.pallas import tpu as pltpu
from jax.experimental.pallas import tpu_sc as plsc

N_NODES = 50000
N_FRAMES = 1600
N_CHAINS = 50
CHAIN_LEN = 32
HID = 48
HEADS = 4
GRU_DIM = 64
C_PAD = 56

N_PAD = 50176
E_TOT = 800000 + N_NODES
E_PAD = 851968
E_ROWS = E_PAD // 128
CH = 1024
RPC = CH // 128
NW = 32
ZR = N_PAD // 16
FPW = N_FRAMES // NW

_SC_PARAMS = pltpu.CompilerParams(use_tc_tiling_on_sc=False)


def _sc_mesh():
    return plsc.VectorSubcoreMesh(core_axis_name="c", subcore_axis_name="s")


def _k1_body(x_ref, w_ref, asm_ref, adm_ref, h1b_ref, as_ref, ad_ref):
    h = jnp.dot(x_ref[...], w_ref[...], preferred_element_type=jnp.float32)
    for j in range(12):
        h1b_ref[j] = h[:, 16 * j:16 * (j + 1)]
    hp = jax.lax.Precision.HIGHEST
    as_ref[...] = jnp.dot(h, asm_ref[...], preferred_element_type=jnp.float32,
                          precision=hp)
    ad_ref[...] = jnp.dot(h, adm_ref[...], preferred_element_type=jnp.float32,
                          precision=hp)


def _k1(x_pad, W1e, asmat, admat):
    grid = N_PAD // CH
    return pl.pallas_call(
        _k1_body,
        grid=(grid,),
        in_specs=[
            pl.BlockSpec((CH, 16), lambda i: (i, 0)),
            pl.BlockSpec((16, 192), lambda i: (0, 0)),
            pl.BlockSpec((192, 16), lambda i: (0, 0)),
            pl.BlockSpec((192, 16), lambda i: (0, 0)),
        ],
        out_specs=[
            pl.BlockSpec((12, CH, 16), lambda i: (0, i, 0)),
            pl.BlockSpec((CH, 16), lambda i: (i, 0)),
            pl.BlockSpec((CH, 16), lambda i: (i, 0)),
        ],
        out_shape=[
            jax.ShapeDtypeStruct((12, N_PAD, 16), jnp.float32),
            jax.ShapeDtypeStruct((N_PAD, 16), jnp.float32),
            jax.ShapeDtypeStruct((N_PAD, 16), jnp.float32),
        ],
    )(x_pad, W1e, asmat, admat)


def _k2_body(o1_ref, dp_ref, b1_ref, w2_ref, asm_ref, adm_ref,
             h2b_ref, as_ref, ad_ref):
    den = dp_ref[0] + dp_ref[1]
    blocks = []
    for j in range(12):
        d = den[:, (j // 3):(j // 3) + 1] + 1e-16
        blocks.append(o1_ref[j] / d)
    y = jnp.concatenate(blocks, axis=1) + b1_ref[...]
    y = jnp.where(y > 0.0, y, jnp.exp(y) - 1.0)
    h2 = jnp.dot(y, w2_ref[...], preferred_element_type=jnp.float32)
    for j in range(3):
        h2b_ref[j] = h2[:, 16 * j:16 * (j + 1)]
    hp = jax.lax.Precision.HIGHEST
    as_ref[...] = jnp.dot(h2, asm_ref[...], preferred_element_type=jnp.float32,
                          precision=hp)
    ad_ref[...] = jnp.dot(h2, adm_ref[...], preferred_element_type=jnp.float32,
                          precision=hp)


def _k2(out1b, dpart1, b1, W2, as2mat, ad2mat):
    grid = N_PAD // CH
    return pl.pallas_call(
        _k2_body,
        grid=(grid,),
        in_specs=[
            pl.BlockSpec((12, CH, 16), lambda i: (0, i, 0)),
            pl.BlockSpec((2, CH, 16), lambda i: (0, i, 0)),
            pl.BlockSpec((1, 192), lambda i: (0, 0)),
            pl.BlockSpec((192, 48), lambda i: (0, 0)),
            pl.BlockSpec((48, 16), lambda i: (0, 0)),
            pl.BlockSpec((48, 16), lambda i: (0, 0)),
        ],
        out_specs=[
            pl.BlockSpec((3, CH, 16), lambda i: (0, i, 0)),
            pl.BlockSpec((CH, 16), lambda i: (i, 0)),
            pl.BlockSpec((CH, 16), lambda i: (i, 0)),
        ],
        out_shape=[
            jax.ShapeDtypeStruct((3, N_PAD, 16), jnp.float32),
            jax.ShapeDtypeStruct((N_PAD, 16), jnp.float32),
            jax.ShapeDtypeStruct((N_PAD, 16), jnp.float32),
        ],
    )(out1b, dpart1, b1, W2, as2mat, ad2mat)


def _make_s1(nheads):
    cpt = E_PAD // NW // CH

    @functools.partial(
        pl.kernel, mesh=_sc_mesh(), compiler_params=_SC_PARAMS,
        out_type=(jax.ShapeDtypeStruct((nheads, E_PAD, 1), jnp.float32),
                  jax.ShapeDtypeStruct((2, N_PAD, 16), jnp.float32)),
        scratch_types=[
            pltpu.VMEM((RPC, 128), jnp.int32),
            pltpu.VMEM((RPC, 128), jnp.int32),
            pltpu.VMEM((CH, 16), jnp.float32),
            pltpu.VMEM((CH, 16), jnp.float32),
            pltpu.VMEM((CH, 16), jnp.float32),
            pltpu.VMEM_SHARED((N_PAD, 16), jnp.float32),
            pltpu.SemaphoreType.DMA,
            pltpu.SemaphoreType.DMA,
        ],
    )
    def s1(src_hbm, dst_hbm, as_hbm, ad_hbm, ex_hbm, dpart_hbm,
           idxs, idxd, asg, adg, exb, acc, sem1, sem2):
        core = lax.axis_index("c")
        sub = lax.axis_index("s")
        wid = core * 16 + sub

        def zb(g, _):
            exb[g, :] = jnp.zeros((16,), jnp.float32)
            return 0
        lax.fori_loop(0, 784, zb, 0)
        for q in range(4):
            pltpu.sync_copy(exb.at[pl.ds(0, 784)],
                            acc.at[pl.ds(sub * ZR + q * 784, 784)])
        plsc.subcore_barrier()

        def chunk(i, _):
            base_row = wid * (cpt * RPC) + i * RPC
            ebase = base_row * 128
            pltpu.sync_copy(src_hbm.at[pl.ds(base_row, RPC)], idxs)
            pltpu.sync_copy(dst_hbm.at[pl.ds(base_row, RPC)], idxd)
            for j in range(RPC):
                pltpu.async_copy(as_hbm.at[idxs.at[j]],
                                 asg.at[pl.ds(j * 128, 128)], sem1)
                pltpu.async_copy(ad_hbm.at[idxd.at[j]],
                                 adg.at[pl.ds(j * 128, 128)], sem2)
            for j in range(RPC):
                pltpu.make_async_copy(as_hbm.at[idxs.at[j]],
                                      asg.at[pl.ds(j * 128, 128)], sem1).wait()
                pltpu.make_async_copy(ad_hbm.at[idxd.at[j]],
                                      adg.at[pl.ds(j * 128, 128)], sem2).wait()

            def vec(g, _):
                e = asg[g, :] + adg[g, :]
                e = jnp.where(e > 0.0, e, 0.2 * e)
                exb[g, :] = jnp.exp(e)
                return 0
            lax.fori_loop(0, CH, vec, 0)
            for h in range(nheads):
                pltpu.sync_copy(exb.at[:, pl.ds(h, 1)],
                                ex_hbm.at[h, pl.ds(ebase, CH)])
            for j in range(RPC):
                pltpu.sync_copy(exb.at[pl.ds(j * 128, 128)],
                                acc.at[idxd.at[j]], add=True)
            return 0
        lax.fori_loop(0, cpt, chunk, 0)
        plsc.subcore_barrier()
        for q in range(4):
            pltpu.sync_copy(acc.at[pl.ds(sub * ZR + q * 784, 784)],
                            dpart_hbm.at[core, pl.ds(sub * ZR + q * 784, 784)])

    return s1


def _make_s3(nblocks, cols0, cols1, head_of):
    cpt = E_PAD // 16 // CH

    @functools.partial(
        pl.kernel, mesh=_sc_mesh(), compiler_params=_SC_PARAMS,
        out_type=jax.ShapeDtypeStruct((nblocks, N_PAD, 16), jnp.float32),
        scratch_types=[
            pltpu.VMEM((RPC, 128), jnp.int32),
            pltpu.VMEM((RPC, 128), jnp.int32),
            pltpu.VMEM((RPC, 128), jnp.float32),
            pltpu.VMEM((CH, 16), jnp.float32),
            pltpu.VMEM((784, 16), jnp.float32),
            pltpu.VMEM_SHARED((N_PAD, 16), jnp.float32),
            pltpu.SemaphoreType.DMA,
        ],
    )
    def s3(src_hbm, dst_hbm, tab_hbm, ex_hbm, out_hbm,
           idxs, idxd, ab, rows, zbuf, acc, sem):
        core = lax.axis_index("c")
        sub = lax.axis_index("s")

        def zb(g, _):
            zbuf[g, :] = jnp.zeros((16,), jnp.float32)
            return 0
        lax.fori_loop(0, 784, zb, 0)

        def do_col(col, head):
            for q in range(4):
                pltpu.sync_copy(zbuf.at[pl.ds(0, 784)],
                                acc.at[pl.ds(sub * ZR + q * 784, 784)])
            plsc.subcore_barrier()

            def chunk(i, _):
                base_row = sub * (cpt * RPC) + i * RPC
                pltpu.sync_copy(src_hbm.at[pl.ds(base_row, RPC)], idxs)
                pltpu.sync_copy(dst_hbm.at[pl.ds(base_row, RPC)], idxd)
                pltpu.sync_copy(ex_hbm.at[head, pl.ds(base_row, RPC)], ab)
                for j in range(RPC):
                    pltpu.async_copy(tab_hbm.at[col].at[idxs.at[j]],
                                     rows.at[pl.ds(j * 128, 128)], sem)
                for j in range(RPC):
                    pltpu.make_async_copy(tab_hbm.at[col].at[idxs.at[j]],
                                          rows.at[pl.ds(j * 128, 128)],
                                          sem).wait()

                def scale(g16, _):
                    a = ab[g16 // 8, pl.ds((g16 % 8) * 16, 16)]
                    for l in range(16):
                        r = g16 * 16 + l
                        rows[r, :] = rows[r, :] * a[l]
                    return 0
                lax.fori_loop(0, 64, scale, 0)
                for j in range(RPC):
                    pltpu.sync_copy(rows.at[pl.ds(j * 128, 128)],
                                    acc.at[idxd.at[j]], add=True)
                return 0
            lax.fori_loop(0, cpt, chunk, 0)
            plsc.subcore_barrier()
            for q in range(4):
                pltpu.sync_copy(acc.at[pl.ds(sub * ZR + q * 784, 784)],
                                out_hbm.at[col, pl.ds(sub * ZR + q * 784, 784)])
            plsc.subcore_barrier()

        for col in cols0:
            @pl.when(core == 0)
            def _(col=col):
                do_col(col, head_of(col))
        for col in cols1:
            @pl.when(core == 1)
            def _(col=col):
                do_col(col, head_of(col))

    return s3


def _make_s4():
    W = 64

    @functools.partial(
        pl.kernel, mesh=_sc_mesh(), compiler_params=_SC_PARAMS,
        out_type=(jax.ShapeDtypeStruct((NW, FPW, 48), jnp.float32),
                  jax.ShapeDtypeStruct((NW, FPW, 48), jnp.float32)),
        scratch_types=[
            pltpu.VMEM((96,), jnp.int32),
            pltpu.VMEM((3, W, 16), jnp.float32),
            pltpu.VMEM((2, W, 16), jnp.float32),
            pltpu.VMEM((8, 16), jnp.float32),
            pltpu.VMEM((FPW, 48), jnp.float32),
            pltpu.VMEM((FPW, 48), jnp.float32),
        ],
    )
    def s4(fs_hbm, o2_hbm, dp_hbm, b2_hbm, sum_hbm, max_hbm,
           fsv, buf, denb, b2b, asum, amax):
        core = lax.axis_index("c")
        sub = lax.axis_index("s")
        wid = core * 16 + sub
        fb = wid * FPW
        fs_base = pl.multiple_of((fb // 8) * 8, 8)
        off = fb - fs_base
        pltpu.sync_copy(fs_hbm.at[pl.ds(fs_base, 96)], fsv)
        pltpu.sync_copy(b2_hbm, b2b)

        def frame(f, _):
            v = fsv[pl.ds(off + f, 16)]
            s0 = v[0]
            s1 = v[1]
            base0 = pl.multiple_of((s0 // 8) * 8, 8)
            nwin = (s1 - base0 + (W - 1)) // W

            def window(w, carry):
                (su0, su1, su2, mx0, mx1, mx2) = carry
                wb = pl.multiple_of(base0 + w * W, 8)
                for cb in range(3):
                    pltpu.sync_copy(o2_hbm.at[cb, pl.ds(wb, W)], buf.at[cb])
                pltpu.sync_copy(dp_hbm.at[0, pl.ds(wb, W)], denb.at[0])
                pltpu.sync_copy(dp_hbm.at[1, pl.ds(wb, W)], denb.at[1])

                def row(r, c2):
                    (t0, t1, t2) = c2
                    node = wb + r
                    c_lo = jnp.where(node >= s0, 1.0, 0.0)
                    c_hi = jnp.where(node < s1, 1.0, 0.0)
                    mf = c_lo * c_hi
                    pen = (mf - 1.0) * 3.0e38
                    d = denb[0, r, :] + denb[1, r, :]
                    dv = 1.0 / (d + 1e-16)
                    inv = dv[0]
                    v0 = buf[0, r, :] * inv + b2b[0, :]
                    v1 = buf[1, r, :] * inv + b2b[1, :]
                    v2 = buf[2, r, :] * inv + b2b[2, :]
                    v0 = jnp.where(v0 > 0.0, v0, jnp.exp(v0) - 1.0)
                    v1 = jnp.where(v1 > 0.0, v1, jnp.exp(v1) - 1.0)
                    v2 = jnp.where(v2 > 0.0, v2, jnp.exp(v2) - 1.0)
                    t0 = (t0[0] + v0 * mf, jnp.maximum(t0[1], v0 * mf + pen))
                    t1 = (t1[0] + v1 * mf, jnp.maximum(t1[1], v1 * mf + pen))
                    t2 = (t2[0] + v2 * mf, jnp.maximum(t2[1], v2 * mf + pen))
                    return (t0, t1, t2)
                (p0, p1, p2) = lax.fori_loop(
                    0, W, row, (((su0, mx0), (su1, mx1), (su2, mx2))))
                return (p0[0], p1[0], p2[0], p0[1], p1[1], p2[1])

            z = jnp.zeros((16,), jnp.float32)
            ninf = z - 3.0e38
            (su0, su1, su2, mx0, mx1, mx2) = lax.fori_loop(
                0, nwin, window, (z, z, z, ninf, ninf, ninf))
            hf = jnp.where(s1 > s0, 1.0, 0.0)
            mx0 = mx0 * hf
            mx1 = mx1 * hf
            mx2 = mx2 * hf
            asum[f, pl.ds(0, 16)] = su0
            asum[f, pl.ds(16, 16)] = su1
            asum[f, pl.ds(32, 16)] = su2
            amax[f, pl.ds(0, 16)] = mx0
            amax[f, pl.ds(16, 16)] = mx1
            amax[f, pl.ds(32, 16)] = mx2
            return 0
        lax.fori_loop(0, FPW, frame, 0)
        pltpu.sync_copy(asum, sum_hbm.at[wid])
        pltpu.sync_copy(amax, max_hbm.at[wid])

    return s4


def _gru_head_body(xsum, xmax, cnt, wi0, wh0, bi0, bh0, wi1, wh1, bi1, bh1,
                   w1, b1, w2, b2, out):
    G = GRU_DIM

    def gates(gi, gh):
        r = jax.nn.sigmoid(gi[:, :G] + gh[:, :G])
        z = jax.nn.sigmoid(gi[:, G:2 * G] + gh[:, G:2 * G])
        n = jnp.tanh(gi[:, 2 * G:] + r * gh[:, 2 * G:3 * G])
        return (1.0 - z) * n + z * gh[:, 3 * G:]

    def step(t, carry):
        h0, h1 = carry
        mean_t = xsum[t] / jnp.maximum(cnt[t], 1.0)
        x_t = jnp.concatenate([mean_t, xmax[t]], axis=1)
        gi0 = jnp.dot(x_t, wi0[...], preferred_element_type=jnp.float32) + bi0[...]
        gh0 = jnp.dot(h0, wh0[...], preferred_element_type=jnp.float32) + bh0[...]
        h0n = gates(gi0, jnp.concatenate([gh0, h0], axis=1))
        gi1 = jnp.dot(h0n, wi1[...], preferred_element_type=jnp.float32) + bi1[...]
        gh1 = jnp.dot(h1, wh1[...], preferred_element_type=jnp.float32) + bh1[...]
        h1n = gates(gi1, jnp.concatenate([gh1, h1], axis=1))
        return h0n, h1n

    h0 = jnp.zeros((C_PAD, GRU_DIM), jnp.float32)
    h0, h1 = lax.fori_loop(0, CHAIN_LEN, step, (h0, h0))
    z = jax.nn.relu(jnp.dot(h1, w1[...], preferred_element_type=jnp.float32) + b1[...])
    out[...] = jnp.dot(z, w2[...], preferred_element_type=jnp.float32) + b2[...]


def _gru_head(xsum, xmax, cnt, Wih0, Whh0, bih0, bhh0, Wih1, Whh1, bih1, bhh1,
              Wh1, bh1, Wh2, bh2):
    args = (xsum, xmax, cnt, Wih0.T, Whh0.T, bih0[None], bhh0[None],
            Wih1.T, Whh1.T, bih1[None], bhh1[None],
            Wh1.T, bh1[None], Wh2.T, bh2[None])
    return pl.pallas_call(
        _gru_head_body,
        out_shape=jax.ShapeDtypeStruct((C_PAD, 1), jnp.float32),
    )(*args)


def _att_mat(att, heads, dim):
    eye = jnp.eye(heads, dtype=att.dtype)
    m = (att[:, :, None] * eye[:, None, :]).reshape(heads * dim, heads)
    return jnp.pad(m, ((0, 0), (0, 16 - heads)))


def kernel(x, edge_index, batch, W1, as1, ad1, b1, W2, as2, ad2, b2,
           Wih0, Whh0, bih0, bhh0, Wih1, Whh1, bih1, bhh1, Wh1, bh1, Wh2, bh2):
    idt = edge_index.dtype
    loop = jnp.arange(N_NODES, dtype=idt)
    pad_e = jnp.full((E_PAD - E_TOT,), N_NODES, idt)
    src = jnp.concatenate([edge_index[0], loop, pad_e]).astype(jnp.int32)
    dst = jnp.concatenate([edge_index[1], loop, pad_e]).astype(jnp.int32)
    src2d = src.reshape(E_ROWS, 128)
    dst2d = dst.reshape(E_ROWS, 128)

    x_pad = jnp.pad(x, ((0, N_PAD - N_NODES), (0, 16 - x.shape[1])))
    W1e = jnp.pad(W1, ((0, 16 - W1.shape[0]), (0, 0)))

    h1b, as1_t, ad1_t = _k1(x_pad, W1e, _att_mat(as1, HEADS, HID),
                            _att_mat(ad1, HEADS, HID))
    ex1, dpart1 = _make_s1(HEADS)(src2d, dst2d, as1_t, ad1_t)
    ex1r = ex1.reshape(HEADS, E_ROWS, 128)
    out1b = _make_s3(12, list(range(6)), list(range(6, 12)),
                     lambda c: c // 3)(src2d, dst2d, h1b, ex1r)

    h2b, as2_t, ad2_t = _k2(out1b, dpart1, b1[None], W2,
                            _att_mat(as2, 1, HID), _att_mat(ad2, 1, HID))
    ex2, dpart2 = _make_s1(1)(src2d, dst2d, as2_t, ad2_t)
    ex2r = ex2.reshape(1, E_ROWS, 128)
    out2b = _make_s3(3, [0, 1], [2], lambda c: 0)(src2d, dst2d, h2b, ex2r)

    fs = jnp.searchsorted(batch, jnp.arange(N_FRAMES + 1, dtype=batch.dtype),
                          side="left").astype(jnp.int32)
    fs_pad = jnp.pad(fs, (0, 1680 - fs.shape[0]), constant_values=N_NODES)
    b2p = jnp.pad(b2.reshape(3, 16), ((0, 5), (0, 0)))
    fsum, fmax = _make_s4()(fs_pad, out2b, dpart2, b2p)

    cnt = jnp.diff(fs).astype(jnp.float32)
    csum = fsum.reshape(N_CHAINS, CHAIN_LEN, 48).transpose(1, 0, 2)
    cmax = fmax.reshape(N_CHAINS, CHAIN_LEN, 48).transpose(1, 0, 2)
    ccnt = jnp.broadcast_to(
        cnt.reshape(N_CHAINS, CHAIN_LEN, 1).transpose(1, 0, 2),
        (CHAIN_LEN, N_CHAINS, 48))
    pad_c = ((0, 0), (0, C_PAD - N_CHAINS), (0, 0))
    csum = jnp.pad(csum, pad_c)
    cmax = jnp.pad(cmax, pad_c)
    ccnt = jnp.pad(ccnt, pad_c, constant_values=1.0)
    logits = _gru_head(csum, cmax, ccnt, Wih0, Whh0, bih0, bhh0,
                       Wih1, Whh1, bih1, bhh1, Wh1, bh1, Wh2, bh2)
    return logits[:N_CHAINS, 0]


# speedup vs baseline: 32.2154x; 3.1363x over previous
import functools

import jax
import jax.numpy as jnp
from jax import lax
from jax.experimental import pallas as pl
from jax.experimental.pallas import tpu as pltpu
from jax.experimental.pallas import tpu_sc as plsc

N_NODES = 50000
N_FRAMES = 1600
N_CHAINS = 50
CHAIN_LEN = 32
HID = 48
HEADS = 4
GRU_DIM = 64
C_PAD = 56

N_PAD = 50176
E_TOT = 800000 + N_NODES
E_PAD = 851968
E_ROWS = E_PAD // 128
CH = 1024
RPC = CH // 128
NW = 32
ZR = N_PAD // 16
FPW = N_FRAMES // NW

_SC_PARAMS = pltpu.CompilerParams(use_tc_tiling_on_sc=False)


def _sc_mesh():
    return plsc.VectorSubcoreMesh(core_axis_name="c", subcore_axis_name="s")


def _k1_body(x_ref, w_ref, asm_ref, adm_ref, h1b_ref, as_ref, ad_ref):
    h = jnp.dot(x_ref[...], w_ref[...], preferred_element_type=jnp.float32)
    for j in range(12):
        h1b_ref[j] = h[:, 16 * j:16 * (j + 1)]
    hp = jax.lax.Precision.HIGHEST
    as_ref[...] = jnp.dot(h, asm_ref[...], preferred_element_type=jnp.float32,
                          precision=hp)
    ad_ref[...] = jnp.dot(h, adm_ref[...], preferred_element_type=jnp.float32,
                          precision=hp)


def _k1(x_pad, W1e, asmat, admat):
    grid = N_PAD // CH
    return pl.pallas_call(
        _k1_body,
        grid=(grid,),
        in_specs=[
            pl.BlockSpec((CH, 16), lambda i: (i, 0)),
            pl.BlockSpec((16, 192), lambda i: (0, 0)),
            pl.BlockSpec((192, 16), lambda i: (0, 0)),
            pl.BlockSpec((192, 16), lambda i: (0, 0)),
        ],
        out_specs=[
            pl.BlockSpec((12, CH, 16), lambda i: (0, i, 0)),
            pl.BlockSpec((CH, 16), lambda i: (i, 0)),
            pl.BlockSpec((CH, 16), lambda i: (i, 0)),
        ],
        out_shape=[
            jax.ShapeDtypeStruct((12, N_PAD, 16), jnp.float32),
            jax.ShapeDtypeStruct((N_PAD, 16), jnp.float32),
            jax.ShapeDtypeStruct((N_PAD, 16), jnp.float32),
        ],
    )(x_pad, W1e, asmat, admat)


def _k2_body(o1_ref, dp_ref, b1_ref, w2_ref, asm_ref, adm_ref,
             h2b_ref, as_ref, ad_ref):
    den = dp_ref[0] + dp_ref[1]
    blocks = []
    for j in range(12):
        d = den[:, (j // 3):(j // 3) + 1] + 1e-16
        blocks.append(o1_ref[j] / d)
    y = jnp.concatenate(blocks, axis=1) + b1_ref[...]
    y = jnp.where(y > 0.0, y, jnp.exp(y) - 1.0)
    h2 = jnp.dot(y, w2_ref[...], preferred_element_type=jnp.float32)
    for j in range(3):
        h2b_ref[j] = h2[:, 16 * j:16 * (j + 1)]
    hp = jax.lax.Precision.HIGHEST
    as_ref[...] = jnp.dot(h2, asm_ref[...], preferred_element_type=jnp.float32,
                          precision=hp)
    ad_ref[...] = jnp.dot(h2, adm_ref[...], preferred_element_type=jnp.float32,
                          precision=hp)


def _k2(out1b, dpart1, b1, W2, as2mat, ad2mat):
    grid = N_PAD // CH
    return pl.pallas_call(
        _k2_body,
        grid=(grid,),
        in_specs=[
            pl.BlockSpec((12, CH, 16), lambda i: (0, i, 0)),
            pl.BlockSpec((2, CH, 16), lambda i: (0, i, 0)),
            pl.BlockSpec((1, 192), lambda i: (0, 0)),
            pl.BlockSpec((192, 48), lambda i: (0, 0)),
            pl.BlockSpec((48, 16), lambda i: (0, 0)),
            pl.BlockSpec((48, 16), lambda i: (0, 0)),
        ],
        out_specs=[
            pl.BlockSpec((3, CH, 16), lambda i: (0, i, 0)),
            pl.BlockSpec((CH, 16), lambda i: (i, 0)),
            pl.BlockSpec((CH, 16), lambda i: (i, 0)),
        ],
        out_shape=[
            jax.ShapeDtypeStruct((3, N_PAD, 16), jnp.float32),
            jax.ShapeDtypeStruct((N_PAD, 16), jnp.float32),
            jax.ShapeDtypeStruct((N_PAD, 16), jnp.float32),
        ],
    )(out1b, dpart1, b1, W2, as2mat, ad2mat)


def _make_s1(nheads):
    cpt = E_PAD // NW // CH

    @functools.partial(
        pl.kernel, mesh=_sc_mesh(), compiler_params=_SC_PARAMS,
        out_type=(jax.ShapeDtypeStruct((nheads, E_ROWS, 128), jnp.float32),
                  jax.ShapeDtypeStruct((2, N_PAD, 16), jnp.float32)),
        scratch_types=[
            pltpu.VMEM((RPC, 128), jnp.int32),
            pltpu.VMEM((RPC, 128), jnp.int32),
            pltpu.VMEM((CH, 16), jnp.float32),
            pltpu.VMEM((CH, 16), jnp.float32),
            pltpu.VMEM((CH, 16), jnp.float32),
            pltpu.VMEM((nheads, RPC, 128), jnp.float32),
            pltpu.VMEM_SHARED((N_PAD, 16), jnp.float32),
            pltpu.SemaphoreType.DMA,
            pltpu.SemaphoreType.DMA,
        ],
    )
    def s1(src_hbm, dst_hbm, as_hbm, ad_hbm, ex_hbm, dpart_hbm,
           idxs, idxd, asg, adg, exb, colb, acc, sem1, sem2):
        core = lax.axis_index("c")
        sub = lax.axis_index("s")
        wid = core * 16 + sub

        def zb(g, _):
            exb[g, :] = jnp.zeros((16,), jnp.float32)
            return 0
        lax.fori_loop(0, 784, zb, 0)
        for q in range(4):
            pltpu.sync_copy(exb.at[pl.ds(0, 784)],
                            acc.at[pl.ds(sub * ZR + q * 784, 784)])
        plsc.subcore_barrier()

        def chunk(i, _):
            base_row = wid * (cpt * RPC) + i * RPC
            pltpu.sync_copy(src_hbm.at[pl.ds(base_row, RPC)], idxs)
            pltpu.sync_copy(dst_hbm.at[pl.ds(base_row, RPC)], idxd)
            for j in range(RPC):
                pltpu.async_copy(as_hbm.at[idxs.at[j]],
                                 asg.at[pl.ds(j * 128, 128)], sem1)
                pltpu.async_copy(ad_hbm.at[idxd.at[j]],
                                 adg.at[pl.ds(j * 128, 128)], sem2)
            for j in range(RPC):
                pltpu.make_async_copy(as_hbm.at[idxs.at[j]],
                                      asg.at[pl.ds(j * 128, 128)], sem1).wait()
                pltpu.make_async_copy(ad_hbm.at[idxd.at[j]],
                                      adg.at[pl.ds(j * 128, 128)], sem2).wait()

            lanef = lax.iota(jnp.int32, 16).astype(jnp.float32)

            def group(g16, _):
                accs = [jnp.zeros((16,), jnp.float32) for _ in range(nheads)]
                for l in range(16):
                    r = g16 * 16 + l
                    e = asg[r, :] + adg[r, :]
                    e = jnp.where(e > 0.0, e, 0.2 * e)
                    ex = jnp.exp(e)
                    exb[r, :] = ex
                    for h in range(nheads):
                        accs[h] = jnp.where(lanef == float(l), ex[h], accs[h])
                for h in range(nheads):
                    colb[h, g16 // 8, pl.ds((g16 % 8) * 16, 16)] = accs[h]
                return 0
            lax.fori_loop(0, CH // 16, group, 0)
            for h in range(nheads):
                pltpu.sync_copy(colb.at[h],
                                ex_hbm.at[h, pl.ds(base_row, RPC)])
            for j in range(RPC):
                pltpu.sync_copy(exb.at[pl.ds(j * 128, 128)],
                                acc.at[idxd.at[j]], add=True)
            return 0
        lax.fori_loop(0, cpt, chunk, 0)
        plsc.subcore_barrier()
        for q in range(4):
            pltpu.sync_copy(acc.at[pl.ds(sub * ZR + q * 784, 784)],
                            dpart_hbm.at[core, pl.ds(sub * ZR + q * 784, 784)])

    return s1


def _make_s3(nblocks, cols0, cols1, head_of):
    cpt = E_PAD // 16 // CH

    @functools.partial(
        pl.kernel, mesh=_sc_mesh(), compiler_params=_SC_PARAMS,
        out_type=jax.ShapeDtypeStruct((nblocks, N_PAD, 16), jnp.float32),
        scratch_types=[
            pltpu.VMEM((RPC, 128), jnp.int32),
            pltpu.VMEM((RPC, 128), jnp.int32),
            pltpu.VMEM((RPC, 128), jnp.float32),
            pltpu.VMEM((CH, 16), jnp.float32),
            pltpu.VMEM((784, 16), jnp.float32),
            pltpu.VMEM_SHARED((N_PAD, 16), jnp.float32),
            pltpu.SemaphoreType.DMA,
        ],
    )
    def s3(src_hbm, dst_hbm, tab_hbm, ex_hbm, out_hbm,
           idxs, idxd, ab, rows, zbuf, acc, sem):
        core = lax.axis_index("c")
        sub = lax.axis_index("s")

        def zb(g, _):
            zbuf[g, :] = jnp.zeros((16,), jnp.float32)
            return 0
        lax.fori_loop(0, 784, zb, 0)

        def do_col(col, head):
            for q in range(4):
                pltpu.sync_copy(zbuf.at[pl.ds(0, 784)],
                                acc.at[pl.ds(sub * ZR + q * 784, 784)])
            plsc.subcore_barrier()

            def chunk(i, _):
                base_row = sub * (cpt * RPC) + i * RPC
                pltpu.sync_copy(src_hbm.at[pl.ds(base_row, RPC)], idxs)
                pltpu.sync_copy(dst_hbm.at[pl.ds(base_row, RPC)], idxd)
                pltpu.sync_copy(ex_hbm.at[head, pl.ds(base_row, RPC)], ab)
                for j in range(RPC):
                    pltpu.async_copy(tab_hbm.at[col].at[idxs.at[j]],
                                     rows.at[pl.ds(j * 128, 128)], sem)
                for j in range(RPC):
                    pltpu.make_async_copy(tab_hbm.at[col].at[idxs.at[j]],
                                          rows.at[pl.ds(j * 128, 128)],
                                          sem).wait()

                def scale(g16, _):
                    a = ab[g16 // 8, pl.ds((g16 % 8) * 16, 16)]
                    for l in range(16):
                        r = g16 * 16 + l
                        rows[r, :] = rows[r, :] * a[l]
                    return 0
                lax.fori_loop(0, 64, scale, 0)
                for j in range(RPC):
                    pltpu.sync_copy(rows.at[pl.ds(j * 128, 128)],
                                    acc.at[idxd.at[j]], add=True)
                return 0
            lax.fori_loop(0, cpt, chunk, 0)
            plsc.subcore_barrier()
            for q in range(4):
                pltpu.sync_copy(acc.at[pl.ds(sub * ZR + q * 784, 784)],
                                out_hbm.at[col, pl.ds(sub * ZR + q * 784, 784)])
            plsc.subcore_barrier()

        for col in cols0:
            @pl.when(core == 0)
            def _(col=col):
                do_col(col, head_of(col))
        for col in cols1:
            @pl.when(core == 1)
            def _(col=col):
                do_col(col, head_of(col))

    return s3


def _make_s4():
    W = 64

    @functools.partial(
        pl.kernel, mesh=_sc_mesh(), compiler_params=_SC_PARAMS,
        out_type=(jax.ShapeDtypeStruct((NW, FPW, 48), jnp.float32),
                  jax.ShapeDtypeStruct((NW, FPW, 48), jnp.float32)),
        scratch_types=[
            pltpu.VMEM((96,), jnp.int32),
            pltpu.VMEM((3, W, 16), jnp.float32),
            pltpu.VMEM((2, W, 16), jnp.float32),
            pltpu.VMEM((8, 16), jnp.float32),
            pltpu.VMEM((FPW, 48), jnp.float32),
            pltpu.VMEM((FPW, 48), jnp.float32),
        ],
    )
    def s4(fs_hbm, o2_hbm, dp_hbm, b2_hbm, sum_hbm, max_hbm,
           fsv, buf, denb, b2b, asum, amax):
        core = lax.axis_index("c")
        sub = lax.axis_index("s")
        wid = core * 16 + sub
        fb = wid * FPW
        fs_base = pl.multiple_of((fb // 8) * 8, 8)
        off = fb - fs_base
        pltpu.sync_copy(fs_hbm.at[pl.ds(fs_base, 96)], fsv)
        pltpu.sync_copy(b2_hbm, b2b)

        def frame(f, _):
            v = fsv[pl.ds(off + f, 16)]
            s0 = v[0]
            s1 = v[1]
            base0 = pl.multiple_of((s0 // 8) * 8, 8)
            nwin = (s1 - base0 + (W - 1)) // W

            def window(w, carry):
                (su0, su1, su2, mx0, mx1, mx2) = carry
                wb = pl.multiple_of(base0 + w * W, 8)
                for cb in range(3):
                    pltpu.sync_copy(o2_hbm.at[cb, pl.ds(wb, W)], buf.at[cb])
                pltpu.sync_copy(dp_hbm.at[0, pl.ds(wb, W)], denb.at[0])
                pltpu.sync_copy(dp_hbm.at[1, pl.ds(wb, W)], denb.at[1])

                def row(r, c2):
                    (t0, t1, t2) = c2
                    node = wb + r
                    c_lo = jnp.where(node >= s0, 1.0, 0.0)
                    c_hi = jnp.where(node < s1, 1.0, 0.0)
                    mf = c_lo * c_hi
                    pen = (mf - 1.0) * 3.0e38
                    d = denb[0, r, :] + denb[1, r, :]
                    dv = 1.0 / (d + 1e-16)
                    inv = dv[0]
                    v0 = buf[0, r, :] * inv + b2b[0, :]
                    v1 = buf[1, r, :] * inv + b2b[1, :]
                    v2 = buf[2, r, :] * inv + b2b[2, :]
                    v0 = jnp.where(v0 > 0.0, v0, jnp.exp(v0) - 1.0)
                    v1 = jnp.where(v1 > 0.0, v1, jnp.exp(v1) - 1.0)
                    v2 = jnp.where(v2 > 0.0, v2, jnp.exp(v2) - 1.0)
                    t0 = (t0[0] + v0 * mf, jnp.maximum(t0[1], v0 * mf + pen))
                    t1 = (t1[0] + v1 * mf, jnp.maximum(t1[1], v1 * mf + pen))
                    t2 = (t2[0] + v2 * mf, jnp.maximum(t2[1], v2 * mf + pen))
                    return (t0, t1, t2)
                (p0, p1, p2) = lax.fori_loop(
                    0, W, row, (((su0, mx0), (su1, mx1), (su2, mx2))))
                return (p0[0], p1[0], p2[0], p0[1], p1[1], p2[1])

            z = jnp.zeros((16,), jnp.float32)
            ninf = z - 3.0e38
            (su0, su1, su2, mx0, mx1, mx2) = lax.fori_loop(
                0, nwin, window, (z, z, z, ninf, ninf, ninf))
            hf = jnp.where(s1 > s0, 1.0, 0.0)
            mx0 = mx0 * hf
            mx1 = mx1 * hf
            mx2 = mx2 * hf
            asum[f, pl.ds(0, 16)] = su0
            asum[f, pl.ds(16, 16)] = su1
            asum[f, pl.ds(32, 16)] = su2
            amax[f, pl.ds(0, 16)] = mx0
            amax[f, pl.ds(16, 16)] = mx1
            amax[f, pl.ds(32, 16)] = mx2
            return 0
        lax.fori_loop(0, FPW, frame, 0)
        pltpu.sync_copy(asum, sum_hbm.at[wid])
        pltpu.sync_copy(amax, max_hbm.at[wid])

    return s4


def _gru_head_body(xsum, xmax, cnt, wi0, wh0, bi0, bh0, wi1, wh1, bi1, bh1,
                   w1, b1, w2, b2, out):
    G = GRU_DIM

    def gates(gi, gh):
        r = jax.nn.sigmoid(gi[:, :G] + gh[:, :G])
        z = jax.nn.sigmoid(gi[:, G:2 * G] + gh[:, G:2 * G])
        n = jnp.tanh(gi[:, 2 * G:] + r * gh[:, 2 * G:3 * G])
        return (1.0 - z) * n + z * gh[:, 3 * G:]

    def step(t, carry):
        h0, h1 = carry
        mean_t = xsum[t] / jnp.maximum(cnt[t], 1.0)
        x_t = jnp.concatenate([mean_t, xmax[t]], axis=1)
        gi0 = jnp.dot(x_t, wi0[...], preferred_element_type=jnp.float32) + bi0[...]
        gh0 = jnp.dot(h0, wh0[...], preferred_element_type=jnp.float32) + bh0[...]
        h0n = gates(gi0, jnp.concatenate([gh0, h0], axis=1))
        gi1 = jnp.dot(h0n, wi1[...], preferred_element_type=jnp.float32) + bi1[...]
        gh1 = jnp.dot(h1, wh1[...], preferred_element_type=jnp.float32) + bh1[...]
        h1n = gates(gi1, jnp.concatenate([gh1, h1], axis=1))
        return h0n, h1n

    h0 = jnp.zeros((C_PAD, GRU_DIM), jnp.float32)
    h0, h1 = lax.fori_loop(0, CHAIN_LEN, step, (h0, h0))
    z = jax.nn.relu(jnp.dot(h1, w1[...], preferred_element_type=jnp.float32) + b1[...])
    out[...] = jnp.dot(z, w2[...], preferred_element_type=jnp.float32) + b2[...]


def _gru_head(xsum, xmax, cnt, Wih0, Whh0, bih0, bhh0, Wih1, Whh1, bih1, bhh1,
              Wh1, bh1, Wh2, bh2):
    args = (xsum, xmax, cnt, Wih0.T, Whh0.T, bih0[None], bhh0[None],
            Wih1.T, Whh1.T, bih1[None], bhh1[None],
            Wh1.T, bh1[None], Wh2.T, bh2[None])
    return pl.pallas_call(
        _gru_head_body,
        out_shape=jax.ShapeDtypeStruct((C_PAD, 1), jnp.float32),
    )(*args)


def _att_mat(att, heads, dim):
    eye = jnp.eye(heads, dtype=att.dtype)
    m = (att[:, :, None] * eye[:, None, :]).reshape(heads * dim, heads)
    return jnp.pad(m, ((0, 0), (0, 16 - heads)))


def kernel(x, edge_index, batch, W1, as1, ad1, b1, W2, as2, ad2, b2,
           Wih0, Whh0, bih0, bhh0, Wih1, Whh1, bih1, bhh1, Wh1, bh1, Wh2, bh2):
    idt = edge_index.dtype
    loop = jnp.arange(N_NODES, dtype=idt)
    pad_e = jnp.full((E_PAD - E_TOT,), N_NODES, idt)
    src = jnp.concatenate([edge_index[0], loop, pad_e]).astype(jnp.int32)
    dst = jnp.concatenate([edge_index[1], loop, pad_e]).astype(jnp.int32)
    src2d = src.reshape(E_ROWS, 128)
    dst2d = dst.reshape(E_ROWS, 128)

    x_pad = jnp.pad(x, ((0, N_PAD - N_NODES), (0, 16 - x.shape[1])))
    W1e = jnp.pad(W1, ((0, 16 - W1.shape[0]), (0, 0)))

    h1b, as1_t, ad1_t = _k1(x_pad, W1e, _att_mat(as1, HEADS, HID),
                            _att_mat(ad1, HEADS, HID))
    ex1, dpart1 = _make_s1(HEADS)(src2d, dst2d, as1_t, ad1_t)
    out1b = _make_s3(12, list(range(6)), list(range(6, 12)),
                     lambda c: c // 3)(src2d, dst2d, h1b, ex1)

    h2b, as2_t, ad2_t = _k2(out1b, dpart1, b1[None], W2,
                            _att_mat(as2, 1, HID), _att_mat(ad2, 1, HID))
    ex2, dpart2 = _make_s1(1)(src2d, dst2d, as2_t, ad2_t)
    out2b = _make_s3(3, [0, 1], [2], lambda c: 0)(src2d, dst2d, h2b, ex2)

    fs = jnp.searchsorted(batch, jnp.arange(N_FRAMES + 1, dtype=batch.dtype),
                          side="left").astype(jnp.int32)
    fs_pad = jnp.pad(fs, (0, 1680 - fs.shape[0]), constant_values=N_NODES)
    b2p = jnp.pad(b2.reshape(3, 16), ((0, 5), (0, 0)))
    fsum, fmax = _make_s4()(fs_pad, out2b, dpart2, b2p)

    cnt = jnp.diff(fs).astype(jnp.float32)
    csum = fsum.reshape(N_CHAINS, CHAIN_LEN, 48).transpose(1, 0, 2)
    cmax = fmax.reshape(N_CHAINS, CHAIN_LEN, 48).transpose(1, 0, 2)
    ccnt = jnp.broadcast_to(
        cnt.reshape(N_CHAINS, CHAIN_LEN, 1).transpose(1, 0, 2),
        (CHAIN_LEN, N_CHAINS, 48))
    pad_c = ((0, 0), (0, C_PAD - N_CHAINS), (0, 0))
    csum = jnp.pad(csum, pad_c)
    cmax = jnp.pad(cmax, pad_c)
    ccnt = jnp.pad(ccnt, pad_c, constant_values=1.0)
    logits = _gru_head(csum, cmax, ccnt, Wih0, Whh0, bih0, bhh0,
                       Wih1, Whh1, bih1, bhh1, Wh1, bh1, Wh2, bh2)
    return logits[:N_CHAINS, 0]
